# baseline (speedup 1.0000x reference)
import functools

import jax
import jax.numpy as jnp
from jax import lax
from jax.experimental import pallas as pl
from jax.experimental.pallas import tpu as pltpu
from jax.experimental.pallas import tpu_sc as plsc

N_NODES = 10000
F = 128
N_EDGES = 320000

NC = 2
NS = 16
NW = NC * NS
MC = 64
MNC = 160
DC = 128
DNC = 80
MIB = 16
MNB = MNC // MIB
NBUF = 4
EPW = MNC * MC
E_PAD = NW * EPW
N_ACC = 10240
RPT = N_ACC // NS

R = 2000
GRID = N_NODES // R

_MESH = plsc.VectorSubcoreMesh(core_axis_name="c", subcore_axis_name="s")


def _deg_body(dst_hbm, deg_out, idx_v, ones_v, zer_v, deg_sh):
    c = lax.axis_index("c")
    s = lax.axis_index("s")
    wid = s * NC + c
    one16 = jnp.ones((16,), jnp.float32)
    zero16 = jnp.zeros((16,), jnp.float32)

    @pl.loop(0, DC // 16)
    def _(i):
        ones_v[pl.ds(i * 16, 16)] = one16

    @pl.loop(0, RPT // 16)
    def _(i):
        zer_v[pl.ds(i * 16, 16)] = zero16

    pltpu.sync_copy(dst_hbm.at[wid], idx_v)
    pltpu.sync_copy(zer_v, deg_sh.at[pl.ds(s * RPT, RPT)])
    plsc.subcore_barrier()

    @pl.loop(0, DNC)
    def _(j):
        pltpu.sync_copy(ones_v, deg_sh.at[idx_v.at[j]], add=True)

    plsc.subcore_barrier()
    pltpu.sync_copy(deg_sh.at[pl.ds(s * RPT, RPT)],
                    deg_out.at[c, pl.ds(s * RPT, RPT)])


_deg_kernel = pl.kernel(
    _deg_body,
    out_type=jax.ShapeDtypeStruct((NC, N_ACC), jnp.float32),
    mesh=_MESH,
    scratch_types=[
        pltpu.VMEM((DNC, DC), jnp.int32),
        pltpu.VMEM((DC,), jnp.float32),
        pltpu.VMEM((RPT,), jnp.float32),
        pltpu.VMEM_SHARED((N_ACC,), jnp.float32),
    ],
)


def _msg_body(src_hbm, dst_hbm, y_hbm, acc_out,
              sidx_v, didx_v, rows_v, zrow_v, acc_sh, sems, ssems):
    c = lax.axis_index("c")
    s = lax.axis_index("s")
    wid = s * NC + c
    zero16 = jnp.zeros((16,), jnp.float32)

    @pl.loop(0, 16 * F // 16)
    def _(i):
        zrow_v[i // 8, pl.ds((i % 8) * 16, 16)] = zero16

    @pl.loop(0, RPT // 16)
    def _(k):
        pltpu.sync_copy(zrow_v, acc_sh.at[pl.ds(s * RPT + k * 16, 16), :])

    plsc.subcore_barrier()

    pltpu.sync_copy(src_hbm.at[wid, 0], sidx_v.at[0])
    pltpu.sync_copy(dst_hbm.at[wid, 0], didx_v.at[0])
    for b in range(3):
        pltpu.async_copy(y_hbm.at[sidx_v.at[0, b]], rows_v.at[b], sems[b])

    def _substep(g, b, first):
        cc = g * NBUF + b
        nf = cc + 3
        bf = (b + 3) % NBUF
        pltpu.make_async_copy(
            y_hbm.at[sidx_v.at[(cc // MIB) % 2, cc % MIB]],
            rows_v.at[b], sems[b]).wait()
        pltpu.make_async_copy(
            rows_v.at[b],
            acc_sh.at[didx_v.at[(cc // MIB) % 2, cc % MIB]],
            ssems[b]).start(add=True)
        if not first:
            pltpu.make_async_copy(
                rows_v.at[bf],
                acc_sh.at[didx_v.at[((cc - 1) // MIB) % 2, (cc - 1) % MIB]],
                ssems[bf]).wait()

        @pl.when(nf < MNC)
        def _():
            pltpu.async_copy(
                y_hbm.at[sidx_v.at[(nf // MIB) % 2, nf % MIB]],
                rows_v.at[bf], sems[bf])

    for b in range(NBUF):
        _substep(0, b, first=(b == 0))

    @pl.loop(1, MNC // NBUF)
    def _(g):
        blk = (g + 3) // 4
        reload = jnp.logical_and(g % 4 == 1, blk < MNB)

        @pl.when(reload)
        def _():
            pltpu.sync_copy(src_hbm.at[wid, blk], sidx_v.at[blk % 2])
            pltpu.sync_copy(dst_hbm.at[wid, blk], didx_v.at[blk % 2])

        for b in range(NBUF):
            _substep(g, b, first=False)

    lc = MNC - 1
    pltpu.make_async_copy(
        rows_v.at[lc % NBUF],
        acc_sh.at[didx_v.at[(lc // MIB) % 2, lc % MIB]],
        ssems[lc % NBUF]).wait()
    plsc.subcore_barrier()
    pltpu.sync_copy(acc_sh.at[pl.ds(s * RPT, RPT), :],
                    acc_out.at[c, pl.ds(s * RPT, RPT), :])


_msg_kernel = pl.kernel(
    _msg_body,
    out_type=jax.ShapeDtypeStruct((NC, N_ACC, F), jnp.float32),
    mesh=_MESH,
    scratch_types=[
        pltpu.VMEM((2, MIB, MC), jnp.int32),
        pltpu.VMEM((2, MIB, MC), jnp.int32),
        pltpu.VMEM((NBUF, MC, F), jnp.float32),
        pltpu.VMEM((16, F), jnp.float32),
        pltpu.VMEM_SHARED((N_ACC, F), jnp.float32),
        [pltpu.SemaphoreType.DMA] * NBUF,
        [pltpu.SemaphoreType.DMA] * NBUF,
    ],
)


def _tc_a_body(emb_ref, w_ref, degp_ref, y_ref, dis_ref):
    deg = degp_ref[0] + degp_ref[1] + 1.0
    dis = lax.rsqrt(deg)
    x = lax.dot_general(emb_ref[...], w_ref[...],
                        (((1,), (1,)), ((), ())),
                        preferred_element_type=jnp.float32)
    dis_ref[...] = dis
    y_ref[...] = x * dis


_tc_a = pl.pallas_call(
    _tc_a_body,
    grid=(GRID,),
    in_specs=[
        pl.BlockSpec((R, F), lambda i: (i, 0)),
        pl.BlockSpec((F, F), lambda i: (0, 0)),
        pl.BlockSpec((NC, R, 1), lambda i: (0, i, 0)),
    ],
    out_specs=[
        pl.BlockSpec((R, F), lambda i: (i, 0)),
        pl.BlockSpec((R, 1), lambda i: (i, 0)),
    ],
    out_shape=[
        jax.ShapeDtypeStruct((N_NODES, F), jnp.float32),
        jax.ShapeDtypeStruct((N_NODES, 1), jnp.float32),
    ],
)


def _tc_b_body(acc_ref, y_ref, dis_ref, b_ref, o_ref):
    o_ref[...] = dis_ref[...] * (acc_ref[0] + acc_ref[1] + y_ref[...]) + b_ref[...]


_tc_b = pl.pallas_call(
    _tc_b_body,
    grid=(GRID,),
    in_specs=[
        pl.BlockSpec((NC, R, F), lambda i: (0, i, 0)),
        pl.BlockSpec((R, F), lambda i: (i, 0)),
        pl.BlockSpec((R, 1), lambda i: (i, 0)),
        pl.BlockSpec((1, F), lambda i: (0, 0)),
    ],
    out_specs=pl.BlockSpec((R, F), lambda i: (i, 0)),
    out_shape=jax.ShapeDtypeStruct((N_NODES, F), jnp.float32),
)


def kernel(embedding, up2down_edge_index, W, b):
    eidx = up2down_edge_index.astype(jnp.int32)
    src, dst = eidx[0], eidx[1]
    npad = E_PAD - src.shape[0]
    ar = jnp.arange(npad, dtype=jnp.int32)
    pad_src = (ar * 131) % N_NODES
    pad_dst = N_NODES + ar % (N_ACC - N_NODES)
    srcp = jnp.concatenate([src, pad_src])
    dstp = jnp.concatenate([dst, pad_dst])
    src4 = srcp.reshape(NW, MNB, MIB, MC)
    dst4 = dstp.reshape(NW, MNB, MIB, MC)
    dst3 = dstp.reshape(NW, DNC, DC)

    deg_parts = _deg_kernel(dst3)
    degp = deg_parts[:, :N_NODES].reshape(NC, N_NODES, 1)
    y, dis = _tc_a(embedding, W, degp)
    acc_parts = _msg_kernel(src4, dst4, y)
    out = _tc_b(acc_parts, y, dis, b.reshape(1, F))
    return out


# speedup vs baseline: 1.0576x; 1.0576x over previous
import functools

import jax
import jax.numpy as jnp
from jax import lax
from jax.experimental import pallas as pl
from jax.experimental.pallas import tpu as pltpu
from jax.experimental.pallas import tpu_sc as plsc

N_NODES = 10000
F = 128
N_EDGES = 320000

NC = 2
NS = 16
NW = NC * NS
MC = 64
MNC = 160
DC = 128
DNC = 80
MIB = 16
MNB = MNC // MIB
NBUF = 4
EPW = MNC * MC
E_PAD = NW * EPW
N_ACC = 10240
RPT = N_ACC // NS

R = 2000
GRID = N_NODES // R

_MESH = plsc.VectorSubcoreMesh(core_axis_name="c", subcore_axis_name="s")


def _deg_body(dst_hbm, deg_out, idx_v, ones_v, zer_v, deg_sh):
    c = lax.axis_index("c")
    s = lax.axis_index("s")
    wid = s * NC + c
    one16 = jnp.ones((16,), jnp.float32)
    zero16 = jnp.zeros((16,), jnp.float32)

    @pl.loop(0, DC // 16)
    def _(i):
        ones_v[pl.ds(i * 16, 16)] = one16

    @pl.loop(0, RPT // 16)
    def _(i):
        zer_v[pl.ds(i * 16, 16)] = zero16

    pltpu.sync_copy(dst_hbm.at[wid], idx_v)
    pltpu.sync_copy(zer_v, deg_sh.at[pl.ds(s * RPT, RPT)])
    plsc.subcore_barrier()

    @pl.loop(0, DNC)
    def _(j):
        pltpu.sync_copy(ones_v, deg_sh.at[idx_v.at[j]], add=True)

    plsc.subcore_barrier()
    pltpu.sync_copy(deg_sh.at[pl.ds(s * RPT, RPT)],
                    deg_out.at[c, pl.ds(s * RPT, RPT)])


_deg_kernel = pl.kernel(
    _deg_body,
    out_type=jax.ShapeDtypeStruct((NC, N_ACC), jnp.float32),
    mesh=_MESH,
    scratch_types=[
        pltpu.VMEM((DNC, DC), jnp.int32),
        pltpu.VMEM((DC,), jnp.float32),
        pltpu.VMEM((RPT,), jnp.float32),
        pltpu.VMEM_SHARED((N_ACC,), jnp.float32),
    ],
)


def _msg_body(src_hbm, dst_hbm, y_hbm, acc_out,
              sidx_v, didx_v, rows_v, zrow_v, acc_sh, sems):
    c = lax.axis_index("c")
    s = lax.axis_index("s")
    wid = s * NC + c
    zero16 = jnp.zeros((16,), jnp.float32)

    @pl.loop(0, 16 * F // 16)
    def _(i):
        zrow_v[i // 8, pl.ds((i % 8) * 16, 16)] = zero16

    @pl.loop(0, RPT // 16)
    def _(k):
        pltpu.sync_copy(zrow_v, acc_sh.at[pl.ds(s * RPT + k * 16, 16), :])

    plsc.subcore_barrier()

    pltpu.sync_copy(src_hbm.at[wid, 0], sidx_v.at[0])
    pltpu.sync_copy(dst_hbm.at[wid, 0], didx_v.at[0])
    for b in range(NBUF):
        pltpu.async_copy(y_hbm.at[sidx_v.at[0, b]], rows_v.at[b], sems[b])

    @pl.loop(0, MNC // NBUF)
    def _(g):
        blk = (g + 3) // 4
        reload = jnp.logical_and(g % 4 == 1, blk < MNB)

        @pl.when(reload)
        def _():
            pltpu.sync_copy(src_hbm.at[wid, blk], sidx_v.at[blk % 2])
            pltpu.sync_copy(dst_hbm.at[wid, blk], didx_v.at[blk % 2])

        for b in range(NBUF):
            cc = g * NBUF + b
            pltpu.make_async_copy(
                y_hbm.at[sidx_v.at[(cc // MIB) % 2, cc % MIB]],
                rows_v.at[b], sems[b]).wait()
            pltpu.sync_copy(
                rows_v.at[b],
                acc_sh.at[didx_v.at[(cc // MIB) % 2, cc % MIB]], add=True)
            nc = cc + NBUF

            @pl.when(nc < MNC)
            def _():
                pltpu.async_copy(
                    y_hbm.at[sidx_v.at[(nc // MIB) % 2, nc % MIB]],
                    rows_v.at[b], sems[b])

    plsc.subcore_barrier()
    pltpu.sync_copy(acc_sh.at[pl.ds(s * RPT, RPT), :],
                    acc_out.at[c, pl.ds(s * RPT, RPT), :])


_msg_kernel = pl.kernel(
    _msg_body,
    out_type=jax.ShapeDtypeStruct((NC, N_ACC, F), jnp.float32),
    mesh=_MESH,
    scratch_types=[
        pltpu.VMEM((2, MIB, MC), jnp.int32),
        pltpu.VMEM((2, MIB, MC), jnp.int32),
        pltpu.VMEM((NBUF, MC, F), jnp.float32),
        pltpu.VMEM((16, F), jnp.float32),
        pltpu.VMEM_SHARED((N_ACC, F), jnp.float32),
        [pltpu.SemaphoreType.DMA] * NBUF,
    ],
)


def _tc_a_body(emb_ref, w_ref, degp_ref, y_ref, dis_ref):
    deg = degp_ref[0] + degp_ref[1] + 1.0
    dis = lax.rsqrt(deg)
    x = lax.dot_general(emb_ref[...], w_ref[...],
                        (((1,), (1,)), ((), ())),
                        preferred_element_type=jnp.float32)
    dis_ref[...] = dis
    y_ref[...] = x * dis


_tc_a = pl.pallas_call(
    _tc_a_body,
    grid=(GRID,),
    in_specs=[
        pl.BlockSpec((R, F), lambda i: (i, 0)),
        pl.BlockSpec((F, F), lambda i: (0, 0)),
        pl.BlockSpec((NC, R, 1), lambda i: (0, i, 0)),
    ],
    out_specs=[
        pl.BlockSpec((R, F), lambda i: (i, 0)),
        pl.BlockSpec((R, 1), lambda i: (i, 0)),
    ],
    out_shape=[
        jax.ShapeDtypeStruct((N_NODES, F), jnp.float32),
        jax.ShapeDtypeStruct((N_NODES, 1), jnp.float32),
    ],
)


def _tc_b_body(acc_ref, y_ref, dis_ref, b_ref, o_ref):
    o_ref[...] = dis_ref[...] * (acc_ref[0] + acc_ref[1] + y_ref[...]) + b_ref[...]


_tc_b = pl.pallas_call(
    _tc_b_body,
    grid=(GRID,),
    in_specs=[
        pl.BlockSpec((NC, R, F), lambda i: (0, i, 0)),
        pl.BlockSpec((R, F), lambda i: (i, 0)),
        pl.BlockSpec((R, 1), lambda i: (i, 0)),
        pl.BlockSpec((1, F), lambda i: (0, 0)),
    ],
    out_specs=pl.BlockSpec((R, F), lambda i: (i, 0)),
    out_shape=jax.ShapeDtypeStruct((N_NODES, F), jnp.float32),
)


def kernel(embedding, up2down_edge_index, W, b):
    eidx = up2down_edge_index.astype(jnp.int32)
    src, dst = eidx[0], eidx[1]
    npad = E_PAD - src.shape[0]
    ar = jnp.arange(npad, dtype=jnp.int32)
    pad_src = (ar * 131) % N_NODES
    pad_dst = N_NODES + ar % (N_ACC - N_NODES)
    srcp = jnp.concatenate([src, pad_src])
    dstp = jnp.concatenate([dst, pad_dst])
    src4 = srcp.reshape(NW, MNB, MIB, MC)
    dst4 = dstp.reshape(NW, MNB, MIB, MC)
    dst3 = dstp.reshape(NW, DNC, DC)

    deg_parts = _deg_kernel(dst3)
    degp = deg_parts[:, :N_NODES].reshape(NC, N_NODES, 1)
    y, dis = _tc_a(embedding, W, degp)
    acc_parts = _msg_kernel(src4, dst4, y)
    out = _tc_b(acc_parts, y, dis, b.reshape(1, F))
    return out


# speedup vs baseline: 1.0903x; 1.0310x over previous
import functools

import jax
import jax.numpy as jnp
from jax import lax
from jax.experimental import pallas as pl
from jax.experimental.pallas import tpu as pltpu
from jax.experimental.pallas import tpu_sc as plsc

N_NODES = 10000
F = 128
N_EDGES = 320000

NC = 2
NS = 16
NW = NC * NS
MC = 64
MNC = 160
DC = 128
DNC = 80
MIB = 16
MNB = MNC // MIB
NBUF = 4
EPW = MNC * MC
E_PAD = NW * EPW
N_ACC = 10240
RPT = N_ACC // NS

R = 2000
GRID = N_NODES // R

_MESH = plsc.VectorSubcoreMesh(core_axis_name="c", subcore_axis_name="s")


def _deg_body(ei_hbm, deg_out, idx_v, ones_v, zer_v, deg_sh):
    c = lax.axis_index("c")
    s = lax.axis_index("s")
    wid = s * NC + c
    one16 = jnp.ones((16,), jnp.float32)
    zero16 = jnp.zeros((16,), jnp.float32)

    @pl.loop(0, DC // 16)
    def _(i):
        ones_v[pl.ds(i * 16, 16)] = one16

    @pl.loop(0, RPT // 16)
    def _(i):
        zer_v[pl.ds(i * 16, 16)] = zero16

    pltpu.sync_copy(ei_hbm.at[1, wid], idx_v)
    pltpu.sync_copy(zer_v, deg_sh.at[pl.ds(s * RPT, RPT)])
    plsc.subcore_barrier()

    @pl.loop(0, DNC)
    def _(j):
        pltpu.sync_copy(ones_v, deg_sh.at[idx_v.at[j]], add=True)

    plsc.subcore_barrier()
    pltpu.sync_copy(deg_sh.at[pl.ds(s * RPT, RPT)],
                    deg_out.at[c, pl.ds(s * RPT, RPT)])


_deg_kernel = pl.kernel(
    _deg_body,
    out_type=jax.ShapeDtypeStruct((NC, N_ACC), jnp.float32),
    mesh=_MESH,
    scratch_types=[
        pltpu.VMEM((DNC, DC), jnp.int32),
        pltpu.VMEM((DC,), jnp.float32),
        pltpu.VMEM((RPT,), jnp.float32),
        pltpu.VMEM_SHARED((N_ACC,), jnp.float32),
    ],
)


def _msg_body(ei_hbm, y_hbm, acc_out,
              sidx_v, didx_v, rows_v, zrow_v, acc_sh, sems):
    c = lax.axis_index("c")
    s = lax.axis_index("s")
    wid = s * NC + c
    zero16 = jnp.zeros((16,), jnp.float32)

    @pl.loop(0, 16 * F // 16)
    def _(i):
        zrow_v[i // 8, pl.ds((i % 8) * 16, 16)] = zero16

    @pl.loop(0, RPT // 16)
    def _(k):
        pltpu.sync_copy(zrow_v, acc_sh.at[pl.ds(s * RPT + k * 16, 16), :])

    plsc.subcore_barrier()

    pltpu.sync_copy(ei_hbm.at[0, wid, 0], sidx_v.at[0])
    pltpu.sync_copy(ei_hbm.at[1, wid, 0], didx_v.at[0])
    for b in range(NBUF):
        pltpu.async_copy(y_hbm.at[sidx_v.at[0, b]], rows_v.at[b], sems[b])

    @pl.loop(0, MNC // NBUF)
    def _(g):
        blk = (g + 3) // 4
        reload = jnp.logical_and(g % 4 == 1, blk < MNB)

        @pl.when(reload)
        def _():
            pltpu.sync_copy(ei_hbm.at[0, wid, blk], sidx_v.at[blk % 2])
            pltpu.sync_copy(ei_hbm.at[1, wid, blk], didx_v.at[blk % 2])

        for b in range(NBUF):
            cc = g * NBUF + b
            pltpu.make_async_copy(
                y_hbm.at[sidx_v.at[(cc // MIB) % 2, cc % MIB]],
                rows_v.at[b], sems[b]).wait()
            pltpu.sync_copy(
                rows_v.at[b],
                acc_sh.at[didx_v.at[(cc // MIB) % 2, cc % MIB]], add=True)
            nc = cc + NBUF

            @pl.when(nc < MNC)
            def _():
                pltpu.async_copy(
                    y_hbm.at[sidx_v.at[(nc // MIB) % 2, nc % MIB]],
                    rows_v.at[b], sems[b])

    plsc.subcore_barrier()
    pltpu.sync_copy(acc_sh.at[pl.ds(s * RPT, RPT), :],
                    acc_out.at[c, pl.ds(s * RPT, RPT), :])


_msg_kernel = pl.kernel(
    _msg_body,
    out_type=jax.ShapeDtypeStruct((NC, N_ACC, F), jnp.float32),
    mesh=_MESH,
    scratch_types=[
        pltpu.VMEM((2, MIB, MC), jnp.int32),
        pltpu.VMEM((2, MIB, MC), jnp.int32),
        pltpu.VMEM((NBUF, MC, F), jnp.float32),
        pltpu.VMEM((16, F), jnp.float32),
        pltpu.VMEM_SHARED((N_ACC, F), jnp.float32),
        [pltpu.SemaphoreType.DMA] * NBUF,
    ],
)


def _tc_a_body(emb_ref, w_ref, degp_ref, y_ref, dis_ref):
    deg = degp_ref[0] + degp_ref[1] + 1.0
    dis = lax.rsqrt(deg)
    x = lax.dot_general(emb_ref[...], w_ref[...],
                        (((1,), (1,)), ((), ())),
                        preferred_element_type=jnp.float32)
    dis_ref[...] = dis
    y_ref[...] = x * dis


_tc_a = pl.pallas_call(
    _tc_a_body,
    grid=(GRID,),
    in_specs=[
        pl.BlockSpec((R, F), lambda i: (i, 0)),
        pl.BlockSpec((F, F), lambda i: (0, 0)),
        pl.BlockSpec((NC, R, 1), lambda i: (0, i, 0)),
    ],
    out_specs=[
        pl.BlockSpec((R, F), lambda i: (i, 0)),
        pl.BlockSpec((R, 1), lambda i: (i, 0)),
    ],
    out_shape=[
        jax.ShapeDtypeStruct((N_NODES, F), jnp.float32),
        jax.ShapeDtypeStruct((N_NODES, 1), jnp.float32),
    ],
)


def _tc_b_body(acc_ref, y_ref, dis_ref, b_ref, o_ref):
    o_ref[...] = dis_ref[...] * (acc_ref[0] + acc_ref[1] + y_ref[...]) + b_ref[...]


_tc_b = pl.pallas_call(
    _tc_b_body,
    grid=(GRID,),
    in_specs=[
        pl.BlockSpec((NC, R, F), lambda i: (0, i, 0)),
        pl.BlockSpec((R, F), lambda i: (i, 0)),
        pl.BlockSpec((R, 1), lambda i: (i, 0)),
        pl.BlockSpec((1, F), lambda i: (0, 0)),
    ],
    out_specs=pl.BlockSpec((R, F), lambda i: (i, 0)),
    out_shape=jax.ShapeDtypeStruct((N_NODES, F), jnp.float32),
)


def kernel(embedding, up2down_edge_index, W, b):
    eidx = up2down_edge_index.astype(jnp.int32)
    npad = E_PAD - eidx.shape[1]
    ar = jnp.arange(npad, dtype=jnp.int32)
    pads = jnp.stack([(ar * 131) % N_NODES,
                      N_NODES + ar % (N_ACC - N_NODES)])
    ei = jnp.concatenate([eidx, pads], axis=1)
    ei5 = ei.reshape(2, NW, MNB, MIB, MC)
    ei3 = ei.reshape(2, NW, DNC, DC)

    deg_parts = _deg_kernel(ei3)
    degp = deg_parts[:, :N_NODES].reshape(NC, N_NODES, 1)
    y, dis = _tc_a(embedding, W, degp)
    acc_parts = _msg_kernel(ei5, y)
    out = _tc_b(acc_parts, y, dis, b.reshape(1, F))
    return out


# speedup vs baseline: 1.1160x; 1.0235x over previous
import functools

import jax
import jax.numpy as jnp
from jax import lax
from jax.experimental import pallas as pl
from jax.experimental.pallas import tpu as pltpu
from jax.experimental.pallas import tpu_sc as plsc

N_NODES = 10000
F = 128
N_EDGES = 320000

NC = 2
NS = 16
NW = NC * NS
MC = 64
MNC = 160
DC = 128
DNC = 80
MIB = 16
MNB = MNC // MIB
NBUF = 4
EPW = MNC * MC
E_PAD = NW * EPW
N_ACC = 10240
RPT = N_ACC // NS

R = 2000
GRID = N_NODES // R

_MESH = plsc.VectorSubcoreMesh(core_axis_name="c", subcore_axis_name="s")


def _deg_body(ei_hbm, deg_out, idx_v, ones_v, zer_v, deg_sh):
    c = lax.axis_index("c")
    s = lax.axis_index("s")
    wid = s * NC + c
    one16 = jnp.ones((16,), jnp.float32)
    zero16 = jnp.zeros((16,), jnp.float32)

    @pl.loop(0, DC // 16)
    def _(i):
        ones_v[pl.ds(i * 16, 16)] = one16

    @pl.loop(0, RPT // 16)
    def _(i):
        zer_v[pl.ds(i * 16, 16)] = zero16

    pltpu.sync_copy(ei_hbm.at[1, wid], idx_v)
    pltpu.sync_copy(zer_v, deg_sh.at[pl.ds(s * RPT, RPT)])
    plsc.subcore_barrier()

    @pl.loop(0, DNC)
    def _(j):
        pltpu.sync_copy(ones_v, deg_sh.at[idx_v.at[j]], add=True)

    plsc.subcore_barrier()
    pltpu.sync_copy(deg_sh.at[pl.ds(s * RPT, RPT)],
                    deg_out.at[c, pl.ds(s * RPT, RPT)])


_deg_kernel = pl.kernel(
    _deg_body,
    out_type=jax.ShapeDtypeStruct((NC, N_ACC), jnp.float32),
    mesh=_MESH,
    scratch_types=[
        pltpu.VMEM((DNC, DC), jnp.int32),
        pltpu.VMEM((DC,), jnp.float32),
        pltpu.VMEM((RPT,), jnp.float32),
        pltpu.VMEM_SHARED((N_ACC,), jnp.float32),
    ],
)


def _msg_body(ei_hbm, y_hbm, acc_out,
              sidx_v, didx_v, rows_v, zrow_v, acc_sh, sems):
    c = lax.axis_index("c")
    s = lax.axis_index("s")
    wid = s * NC + c
    zero16 = jnp.zeros((16,), jnp.float32)

    @pl.loop(0, 16 * F // 16)
    def _(i):
        zrow_v[i // 8, pl.ds((i % 8) * 16, 16)] = zero16

    @pl.loop(0, RPT // 16)
    def _(k):
        pltpu.sync_copy(zrow_v, acc_sh.at[pl.ds(s * RPT + k * 16, 16), :])

    plsc.subcore_barrier()

    pltpu.sync_copy(ei_hbm.at[0, wid, 0], sidx_v.at[0])
    pltpu.sync_copy(ei_hbm.at[1, wid, 0], didx_v.at[0])
    for b in range(NBUF):
        pltpu.async_copy(y_hbm.at[sidx_v.at[0, b]], rows_v.at[b], sems[b])

    @pl.loop(0, MNC // NBUF)
    def _(g):
        blk = (g + 3) // 4
        reload = jnp.logical_and(g % 4 == 1, blk < MNB)

        @pl.when(reload)
        def _():
            pltpu.sync_copy(ei_hbm.at[0, wid, blk], sidx_v.at[blk % 2])
            pltpu.sync_copy(ei_hbm.at[1, wid, blk], didx_v.at[blk % 2])

        for b in range(NBUF):
            cc = g * NBUF + b
            pltpu.make_async_copy(
                y_hbm.at[sidx_v.at[(cc // MIB) % 2, cc % MIB]],
                rows_v.at[b], sems[b]).wait()
            pltpu.sync_copy(
                rows_v.at[b],
                acc_sh.at[didx_v.at[(cc // MIB) % 2, cc % MIB]], add=True)
            nc = cc + NBUF

            @pl.when(nc < MNC)
            def _():
                pltpu.async_copy(
                    y_hbm.at[sidx_v.at[(nc // MIB) % 2, nc % MIB]],
                    rows_v.at[b], sems[b])

    plsc.subcore_barrier()
    pltpu.sync_copy(acc_sh.at[pl.ds(s * RPT, RPT), :],
                    acc_out.at[c, pl.ds(s * RPT, RPT), :])


_msg_kernel = pl.kernel(
    _msg_body,
    out_type=jax.ShapeDtypeStruct((NC, N_ACC, F), jnp.float32),
    mesh=_MESH,
    scratch_types=[
        pltpu.VMEM((2, MIB, MC), jnp.int32),
        pltpu.VMEM((2, MIB, MC), jnp.int32),
        pltpu.VMEM((NBUF, MC, F), jnp.float32),
        pltpu.VMEM((16, F), jnp.float32),
        pltpu.VMEM_SHARED((N_ACC, F), jnp.float32),
        [pltpu.SemaphoreType.DMA] * NBUF,
    ],
)


def _dis_of(degp):
    return lax.rsqrt(degp[:, 0:1] + degp[:, 1:2] + 1.0)


def _tc_a_body(emb_ref, w_ref, degp_ref, y_ref):
    dis = _dis_of(degp_ref[...])
    x = lax.dot_general(emb_ref[...], w_ref[...],
                        (((1,), (1,)), ((), ())),
                        preferred_element_type=jnp.float32)
    y_ref[...] = x * dis


_tc_a = pl.pallas_call(
    _tc_a_body,
    grid=(GRID,),
    in_specs=[
        pl.BlockSpec((R, F), lambda i: (i, 0)),
        pl.BlockSpec((F, F), lambda i: (0, 0)),
        pl.BlockSpec((R, NC), lambda i: (i, 0)),
    ],
    out_specs=pl.BlockSpec((R, F), lambda i: (i, 0)),
    out_shape=jax.ShapeDtypeStruct((N_NODES, F), jnp.float32),
)


def _tc_b_body(acc_ref, y_ref, degp_ref, b_ref, o_ref):
    dis = _dis_of(degp_ref[...])
    o_ref[...] = dis * (acc_ref[0] + acc_ref[1] + y_ref[...]) + b_ref[...]


_tc_b = pl.pallas_call(
    _tc_b_body,
    grid=(GRID,),
    in_specs=[
        pl.BlockSpec((NC, R, F), lambda i: (0, i, 0)),
        pl.BlockSpec((R, F), lambda i: (i, 0)),
        pl.BlockSpec((R, NC), lambda i: (i, 0)),
        pl.BlockSpec((1, F), lambda i: (0, 0)),
    ],
    out_specs=pl.BlockSpec((R, F), lambda i: (i, 0)),
    out_shape=jax.ShapeDtypeStruct((N_NODES, F), jnp.float32),
)


def kernel(embedding, up2down_edge_index, W, b):
    eidx = up2down_edge_index.astype(jnp.int32)
    npad = E_PAD - eidx.shape[1]
    ar = jnp.arange(npad, dtype=jnp.int32)
    pads = jnp.stack([(ar * 131) % N_NODES,
                      N_NODES + ar % (N_ACC - N_NODES)])
    ei = jnp.concatenate([eidx, pads], axis=1)
    ei5 = ei.reshape(2, NW, MNB, MIB, MC)
    ei3 = ei.reshape(2, NW, DNC, DC)

    deg_parts = _deg_kernel(ei3)
    degp = deg_parts.T
    y = _tc_a(embedding, W, degp)
    acc_parts = _msg_kernel(ei5, y)
    out = _tc_b(acc_parts, y, degp, b.reshape(1, F))
    return out
